# baseline (speedup 1.0000x reference)
import dataclasses
import functools

import jax
import jax.numpy as jnp
from jax import lax
from jax.experimental import pallas as pl
from jax.experimental.pallas import tpu as pltpu
from jax.experimental.pallas import tpu_sc as plsc

N = 10000
E = 320000
D = 128

NC = 2
NS = 16
NW = NC * NS
FPW = D // NW
EPW = E // NW
LANES = 16
K3_CHUNK = 8000

_sc_mesh = plsc.VectorSubcoreMesh(core_axis_name="c", subcore_axis_name="s")

_sc_params = pltpu.CompilerParams()
if "needs_layout_passes" in pltpu.CompilerParams.__dataclass_fields__:
    _sc_params = dataclasses.replace(_sc_params, needs_layout_passes=False)


def _wid():
    return lax.axis_index("c") * NS + lax.axis_index("s")


@functools.partial(
    pl.kernel,
    out_type=jax.ShapeDtypeStruct((NW * N,), jnp.float32),
    mesh=_sc_mesh,
    scratch_types=[
        pltpu.VMEM((EPW,), jnp.int32),
        pltpu.VMEM((EPW,), jnp.float32),
        pltpu.VMEM((N,), jnp.float32),
    ],
    compiler_params=_sc_params,
)
def _k1_deg(dst_hbm, w_hbm, out_hbm, dst_v, w_v, deg_v):
    wid = _wid()
    base = wid * EPW
    pltpu.sync_copy(dst_hbm.at[pl.ds(base, EPW)], dst_v)
    pltpu.sync_copy(w_hbm.at[pl.ds(base, EPW)], w_v)

    zeros = jnp.zeros((LANES,), jnp.float32)

    @pl.loop(0, N, step=LANES)
    def _(i):
        deg_v[pl.ds(i, LANES)] = zeros

    @plsc.parallel_loop(0, EPW, step=LANES, unroll=4)
    def _(i):
        d16 = dst_v[pl.ds(i, LANES)]
        w16 = w_v[pl.ds(i, LANES)]
        plsc.addupdate_scatter(deg_v, [d16], w16)

    pltpu.sync_copy(deg_v, out_hbm.at[pl.ds(wid * N, N)])


def _k2a_body(x_ref, w_ref, xwt_ref):
    xwt_ref[...] = lax.dot_general(
        w_ref[...], x_ref[...],
        dimension_numbers=(((0,), (1,)), ((), ())),
        preferred_element_type=jnp.float32,
    )


def _k2a(x, W2):
    return pl.pallas_call(
        _k2a_body,
        in_specs=[
            pl.BlockSpec((N, D), lambda: (0, 0)),
            pl.BlockSpec((D, D), lambda: (0, 0)),
        ],
        out_specs=pl.BlockSpec((D, N), lambda: (0, 0)),
        out_shape=jax.ShapeDtypeStruct((D, N), jnp.float32),
    )(x, W2)


def _k2b_body(xwt_ref, degp_ref, ytp_ref, dinv_ref):
    deg = 1.0 + jnp.sum(degp_ref[...], axis=0)
    dinv = lax.rsqrt(deg)
    hi = xwt_ref[: D // 2, :] * dinv[None, :]
    lo = xwt_ref[D // 2 :, :] * dinv[None, :]
    hb = lax.convert_element_type(
        lax.bitcast_convert_type(lax.convert_element_type(hi, jnp.bfloat16),
                                 jnp.uint16), jnp.uint32)
    lb = lax.convert_element_type(
        lax.bitcast_convert_type(lax.convert_element_type(lo, jnp.bfloat16),
                                 jnp.uint16), jnp.uint32)
    word = (hb << 16) | lb
    ytp_ref[...] = lax.bitcast_convert_type(word, jnp.int32)
    dinv_ref[...] = dinv[None, :]


def _k2b(xwt2, deg_part):
    return pl.pallas_call(
        _k2b_body,
        in_specs=[
            pl.BlockSpec((D, N), lambda: (0, 0)),
            pl.BlockSpec((NW, N), lambda: (0, 0)),
        ],
        out_specs=[
            pl.BlockSpec((D // 2, N), lambda: (0, 0)),
            pl.BlockSpec((1, N), lambda: (0, 0)),
        ],
        out_shape=[
            jax.ShapeDtypeStruct((D // 2, N), jnp.int32),
            jax.ShapeDtypeStruct((1, N), jnp.float32),
        ],
    )(xwt2, deg_part)


NCH = E // K3_CHUNK


@functools.partial(
    pl.kernel,
    out_type=jax.ShapeDtypeStruct((D * N,), jnp.float32),
    mesh=_sc_mesh,
    scratch_types=[
        pltpu.VMEM((2 * N,), jnp.int32),
        pltpu.VMEM((FPW * N,), jnp.float32),
        pltpu.VMEM((N,), jnp.float32),
        pltpu.VMEM((2 * K3_CHUNK,), jnp.int32),
        pltpu.VMEM((2 * K3_CHUNK,), jnp.float32),
        pltpu.SemaphoreType.DMA,
        pltpu.SemaphoreType.DMA,
        pltpu.SemaphoreType.DMA,
        pltpu.SemaphoreType.DMA,
    ],
    compiler_params=_sc_params,
)
def _k3_msg(ytp_hbm, dinv_hbm, p_hbm, w_hbm, out_hbm,
            ytp_v, acc_v, dinv_v, p_v, w_v, sp0, sw0, sp1, sw1):
    wid = _wid()
    f0 = wid * FPW
    mask_hi = jnp.int32(-65536)

    def start(c, b):
        psem, wsem = (sp0, sw0) if b == 0 else (sp1, sw1)
        pltpu.async_copy(p_hbm.at[pl.ds(c * K3_CHUNK, K3_CHUNK)],
                         p_v.at[pl.ds(b * K3_CHUNK, K3_CHUNK)], psem)
        pltpu.async_copy(w_hbm.at[pl.ds(c * K3_CHUNK, K3_CHUNK)],
                         w_v.at[pl.ds(b * K3_CHUNK, K3_CHUNK)], wsem)

    def wait(b):
        psem, wsem = (sp0, sw0) if b == 0 else (sp1, sw1)
        pltpu.make_async_copy(p_hbm.at[pl.ds(0, K3_CHUNK)],
                              p_v.at[pl.ds(b * K3_CHUNK, K3_CHUNK)], psem).wait()
        pltpu.make_async_copy(w_hbm.at[pl.ds(0, K3_CHUNK)],
                              w_v.at[pl.ds(b * K3_CHUNK, K3_CHUNK)], wsem).wait()

    def process(b):
        base = b * K3_CHUNK

        @plsc.parallel_loop(0, K3_CHUNK, step=LANES, unroll=8)
        def _(i):
            p16 = p_v[pl.ds(base + i, LANES)]
            w16 = w_v[pl.ds(base + i, LANES)]
            s16 = lax.shift_right_logical(p16, 14)
            d16 = p16 & 16383
            for r in range(2):
                wp = plsc.load_gather(ytp_v, [s16 + (r * N)])
                hi = plsc.bitcast(wp & mask_hi, jnp.float32)
                lo = plsc.bitcast(lax.shift_left(wp, 16), jnp.float32)
                plsc.addupdate_scatter(acc_v, [d16 + (2 * r * N)], hi * w16)
                plsc.addupdate_scatter(acc_v, [d16 + ((2 * r + 1) * N)], lo * w16)

    start(0, 0)
    start(1, 1)

    for r in range(2):
        pltpu.sync_copy(ytp_hbm.at[pl.ds((2 * wid + r) * N, N)],
                        ytp_v.at[pl.ds(r * N, N)])
    pltpu.sync_copy(dinv_hbm, dinv_v)

    zeros = jnp.zeros((LANES,), jnp.float32)

    @plsc.parallel_loop(0, FPW * N, step=LANES, unroll=4)
    def _(i):
        acc_v[pl.ds(i, LANES)] = zeros

    @pl.loop(0, NCH - 2, step=2)
    def _(c):
        wait(0)
        process(0)
        start(c + 2, 0)
        wait(1)
        process(1)
        start(c + 3, 1)

    wait(0)
    process(0)
    wait(1)
    process(1)

    @plsc.parallel_loop(0, N, step=LANES, unroll=4)
    def _(i):
        dv = dinv_v[pl.ds(i, LANES)]
        for r in range(2):
            wp = ytp_v[pl.ds(r * N + i, LANES)]
            hi = plsc.bitcast(wp & mask_hi, jnp.float32)
            lo = plsc.bitcast(lax.shift_left(wp, 16), jnp.float32)
            j0 = pl.ds(2 * r * N + i, LANES)
            j1 = pl.ds((2 * r + 1) * N + i, LANES)
            acc_v[j0] = dv * (acc_v[j0] + hi)
            acc_v[j1] = dv * (acc_v[j1] + lo)

    for f in range(FPW):
        pltpu.sync_copy(acc_v.at[pl.ds(f * N, N)], out_hbm.at[pl.ds((f0 + f) * N, N)])


def _k4_body(outt_ref, b_ref, out_ref):
    out_ref[...] = outt_ref[...].T + b_ref[...]


def _k4(out_t, b):
    return pl.pallas_call(
        _k4_body,
        in_specs=[
            pl.BlockSpec((D, N), lambda: (0, 0)),
            pl.BlockSpec((1, D), lambda: (0, 0)),
        ],
        out_specs=pl.BlockSpec((N, D), lambda: (0, 0)),
        out_shape=jax.ShapeDtypeStruct((N, D), jnp.float32),
    )(out_t, b.reshape(1, D))


def kernel(x, edge_index, weight, batch, W, b):
    src = edge_index[0]
    dst = edge_index[1]
    packed = (src << 14) | dst
    W2 = jnp.concatenate([W[:, 0::2], W[:, 1::2]], axis=1)
    deg_part = _k1_deg(dst, weight)
    xwt2 = _k2a(x, W2)
    ytp, dinv = _k2b(xwt2, deg_part.reshape(NW, N))
    out_t = _k3_msg(ytp.reshape(D // 2 * N), dinv.reshape(N), packed, weight)
    return _k4(out_t.reshape(D, N), b)


# speedup vs baseline: 1.0187x; 1.0187x over previous
import dataclasses
import functools

import jax
import jax.numpy as jnp
from jax import lax
from jax.experimental import pallas as pl
from jax.experimental.pallas import tpu as pltpu
from jax.experimental.pallas import tpu_sc as plsc

N = 10000
E = 320000
D = 128

NC = 2
NS = 16
NW = NC * NS
FPW = D // NW
EPW = E // NW
LANES = 16
K3_CHUNK = 8000

_sc_mesh = plsc.VectorSubcoreMesh(core_axis_name="c", subcore_axis_name="s")

_sc_params = pltpu.CompilerParams()
if "needs_layout_passes" in pltpu.CompilerParams.__dataclass_fields__:
    _sc_params = dataclasses.replace(_sc_params, needs_layout_passes=False)


def _wid():
    return lax.axis_index("c") * NS + lax.axis_index("s")


@functools.partial(
    pl.kernel,
    out_type=jax.ShapeDtypeStruct((NW * N,), jnp.float32),
    mesh=_sc_mesh,
    scratch_types=[
        pltpu.VMEM((EPW,), jnp.int32),
        pltpu.VMEM((EPW,), jnp.float32),
        pltpu.VMEM((N,), jnp.float32),
    ],
    compiler_params=_sc_params,
)
def _k1_deg(dst_hbm, w_hbm, out_hbm, dst_v, w_v, deg_v):
    wid = _wid()
    base = wid * EPW
    pltpu.sync_copy(dst_hbm.at[pl.ds(base, EPW)], dst_v)
    pltpu.sync_copy(w_hbm.at[pl.ds(base, EPW)], w_v)

    zeros = jnp.zeros((LANES,), jnp.float32)

    @pl.loop(0, N, step=LANES)
    def _(i):
        deg_v[pl.ds(i, LANES)] = zeros

    @plsc.parallel_loop(0, EPW, step=LANES, unroll=4)
    def _(i):
        d16 = dst_v[pl.ds(i, LANES)]
        w16 = w_v[pl.ds(i, LANES)]
        plsc.addupdate_scatter(deg_v, [d16], w16)

    pltpu.sync_copy(deg_v, out_hbm.at[pl.ds(wid * N, N)])


def _k2a_body(x_ref, w_ref, xwt_ref):
    xwt_ref[...] = lax.dot_general(
        w_ref[...], x_ref[...],
        dimension_numbers=(((0,), (1,)), ((), ())),
        preferred_element_type=jnp.float32,
    )


def _k2a(x, W2):
    return pl.pallas_call(
        _k2a_body,
        in_specs=[
            pl.BlockSpec((N, D), lambda: (0, 0)),
            pl.BlockSpec((D, D), lambda: (0, 0)),
        ],
        out_specs=pl.BlockSpec((D, N), lambda: (0, 0)),
        out_shape=jax.ShapeDtypeStruct((D, N), jnp.float32),
    )(x, W2)


def _k2b_body(xwt_ref, degp_ref, ytp_ref, dinv_ref):
    deg = 1.0 + jnp.sum(degp_ref[...], axis=0)
    dinv = lax.rsqrt(deg)
    hi = xwt_ref[: D // 2, :] * dinv[None, :]
    lo = xwt_ref[D // 2 :, :] * dinv[None, :]
    hb = lax.convert_element_type(
        lax.bitcast_convert_type(lax.convert_element_type(hi, jnp.bfloat16),
                                 jnp.uint16), jnp.uint32)
    lb = lax.convert_element_type(
        lax.bitcast_convert_type(lax.convert_element_type(lo, jnp.bfloat16),
                                 jnp.uint16), jnp.uint32)
    word = (hb << 16) | lb
    ytp_ref[...] = lax.bitcast_convert_type(word, jnp.int32)
    dinv_ref[...] = dinv[None, :]


def _k2b(xwt2, deg_part):
    return pl.pallas_call(
        _k2b_body,
        in_specs=[
            pl.BlockSpec((D, N), lambda: (0, 0)),
            pl.BlockSpec((NW, N), lambda: (0, 0)),
        ],
        out_specs=[
            pl.BlockSpec((D // 2, N), lambda: (0, 0)),
            pl.BlockSpec((1, N), lambda: (0, 0)),
        ],
        out_shape=[
            jax.ShapeDtypeStruct((D // 2, N), jnp.int32),
            jax.ShapeDtypeStruct((1, N), jnp.float32),
        ],
    )(xwt2, deg_part)


NCH = E // K3_CHUNK


@functools.partial(
    pl.kernel,
    out_type=jax.ShapeDtypeStruct((D * N,), jnp.float32),
    mesh=_sc_mesh,
    scratch_types=[
        pltpu.VMEM((2 * N,), jnp.int32),
        pltpu.VMEM((FPW * N,), jnp.float32),
        pltpu.VMEM((N,), jnp.float32),
        pltpu.VMEM((2 * K3_CHUNK,), jnp.int32),
        pltpu.VMEM((2 * K3_CHUNK,), jnp.float32),
        pltpu.SemaphoreType.DMA,
        pltpu.SemaphoreType.DMA,
        pltpu.SemaphoreType.DMA,
        pltpu.SemaphoreType.DMA,
    ],
    compiler_params=_sc_params,
)
def _k3_msg(ytp_hbm, dinv_hbm, p_hbm, w_hbm, out_hbm,
            ytp_v, acc_v, dinv_v, p_v, w_v, sp0, sw0, sp1, sw1):
    wid = _wid()
    f0 = wid * FPW
    mask_hi = jnp.int32(-65536)

    def start(c, b):
        psem, wsem = (sp0, sw0) if b == 0 else (sp1, sw1)
        pltpu.async_copy(p_hbm.at[pl.ds(c * K3_CHUNK, K3_CHUNK)],
                         p_v.at[pl.ds(b * K3_CHUNK, K3_CHUNK)], psem)
        pltpu.async_copy(w_hbm.at[pl.ds(c * K3_CHUNK, K3_CHUNK)],
                         w_v.at[pl.ds(b * K3_CHUNK, K3_CHUNK)], wsem)

    def wait(b):
        psem, wsem = (sp0, sw0) if b == 0 else (sp1, sw1)
        pltpu.make_async_copy(p_hbm.at[pl.ds(0, K3_CHUNK)],
                              p_v.at[pl.ds(b * K3_CHUNK, K3_CHUNK)], psem).wait()
        pltpu.make_async_copy(w_hbm.at[pl.ds(0, K3_CHUNK)],
                              w_v.at[pl.ds(b * K3_CHUNK, K3_CHUNK)], wsem).wait()

    def process(b):
        base = b * K3_CHUNK

        @plsc.parallel_loop(0, K3_CHUNK, step=LANES, unroll=4)
        def _(i):
            p16 = p_v[pl.ds(base + i, LANES)]
            w16 = w_v[pl.ds(base + i, LANES)]
            s16 = lax.shift_right_logical(p16, 14)
            d16 = p16 & 16383
            for r in range(2):
                wp = plsc.load_gather(ytp_v, [s16 + (r * N)])
                hi = plsc.bitcast(wp & mask_hi, jnp.float32)
                lo = plsc.bitcast(lax.shift_left(wp, 16), jnp.float32)
                plsc.addupdate_scatter(acc_v, [d16 + (2 * r * N)], hi * w16)
                plsc.addupdate_scatter(acc_v, [d16 + ((2 * r + 1) * N)], lo * w16)

    start(0, 0)
    start(1, 1)

    for r in range(2):
        pltpu.sync_copy(ytp_hbm.at[pl.ds((2 * wid + r) * N, N)],
                        ytp_v.at[pl.ds(r * N, N)])
    pltpu.sync_copy(dinv_hbm, dinv_v)

    zeros = jnp.zeros((LANES,), jnp.float32)

    @plsc.parallel_loop(0, FPW * N, step=LANES, unroll=4)
    def _(i):
        acc_v[pl.ds(i, LANES)] = zeros

    @pl.loop(0, NCH - 2, step=2)
    def _(c):
        wait(0)
        process(0)
        start(c + 2, 0)
        wait(1)
        process(1)
        start(c + 3, 1)

    wait(0)
    process(0)
    wait(1)
    process(1)

    @plsc.parallel_loop(0, N, step=LANES, unroll=4)
    def _(i):
        dv = dinv_v[pl.ds(i, LANES)]
        for r in range(2):
            wp = ytp_v[pl.ds(r * N + i, LANES)]
            hi = plsc.bitcast(wp & mask_hi, jnp.float32)
            lo = plsc.bitcast(lax.shift_left(wp, 16), jnp.float32)
            j0 = pl.ds(2 * r * N + i, LANES)
            j1 = pl.ds((2 * r + 1) * N + i, LANES)
            acc_v[j0] = dv * (acc_v[j0] + hi)
            acc_v[j1] = dv * (acc_v[j1] + lo)

    for f in range(FPW):
        pltpu.sync_copy(acc_v.at[pl.ds(f * N, N)], out_hbm.at[pl.ds((f0 + f) * N, N)])


def _k4_body(outt_ref, b_ref, out_ref):
    out_ref[...] = outt_ref[...].T + b_ref[...]


def _k4(out_t, b):
    return pl.pallas_call(
        _k4_body,
        in_specs=[
            pl.BlockSpec((D, N), lambda: (0, 0)),
            pl.BlockSpec((1, D), lambda: (0, 0)),
        ],
        out_specs=pl.BlockSpec((N, D), lambda: (0, 0)),
        out_shape=jax.ShapeDtypeStruct((N, D), jnp.float32),
    )(out_t, b.reshape(1, D))


def kernel(x, edge_index, weight, batch, W, b):
    src = edge_index[0]
    dst = edge_index[1]
    packed = (src << 14) | dst
    W2 = jnp.concatenate([W[:, 0::2], W[:, 1::2]], axis=1)
    deg_part = _k1_deg(dst, weight)
    xwt2 = _k2a(x, W2)
    ytp, dinv = _k2b(xwt2, deg_part.reshape(NW, N))
    out_t = _k3_msg(ytp.reshape(D // 2 * N), dinv.reshape(N), packed, weight)
    return _k4(out_t.reshape(D, N), b)
